# Initial kernel scaffold; baseline (speedup 1.0000x reference)
import jax
import jax.numpy as jnp
from jax.experimental import pallas as pl


def kernel(feature_values, table):
    raise NotImplementedError("write your pallas kernel here")


# speedup vs baseline: 1.7020x; 1.7020x over previous
import functools

import jax
import jax.numpy as jnp
from jax import lax
from jax.experimental import pallas as pl
from jax.experimental.pallas import tpu as pltpu
from jax.experimental.pallas import tpu_sc as plsc

NUM_BUCKETS = 1000000
EMBED_DIM = 64
LANES = 16
CHUNK = 512


@functools.lru_cache(maxsize=None)
def _make_sc_gather(B: int):
    info = plsc.get_sparse_core_info()
    nc, ns = info.num_cores, info.num_subcores
    nw = nc * ns
    assert B % (8 * nw) == 0
    b_per_w = B // nw
    assert b_per_w % CHUNK == 0
    n_chunks = b_per_w // CHUNK

    mesh = plsc.VectorSubcoreMesh(core_axis_name="c", subcore_axis_name="s")

    @functools.partial(
        pl.kernel,
        mesh=mesh,
        out_type=jax.ShapeDtypeStruct((B, EMBED_DIM), jnp.float32),
        compiler_params=pltpu.CompilerParams(use_tc_tiling_on_sc=False),
        scratch_types=[
            pltpu.VMEM((CHUNK,), jnp.int32),
            pltpu.VMEM((CHUNK,), jnp.int32),
            pltpu.VMEM((CHUNK, EMBED_DIM), jnp.float32),
            pltpu.SemaphoreType.DMA,
        ],
    )
    def sc_gather(fv_hbm, table_hbm, out_hbm, fv_v, idx_v, rows_v, sem):
        wid = lax.axis_index("s") * nc + lax.axis_index("c")
        base = wid * b_per_w

        def chunk_body(c, carry):
            off = base + c * CHUNK
            pltpu.sync_copy(fv_hbm.at[pl.ds(off, CHUNK)], fv_v)

            def mod_body(i, carry2):
                v = fv_v[pl.ds(i * LANES, LANES)]
                idx_v[pl.ds(i * LANES, LANES)] = lax.rem(v, NUM_BUCKETS)
                return carry2

            lax.fori_loop(0, CHUNK // LANES, mod_body, 0, unroll=4)
            pltpu.async_copy(table_hbm.at[idx_v], rows_v, sem).wait()
            pltpu.sync_copy(rows_v, out_hbm.at[pl.ds(off, CHUNK)])
            return carry

        lax.fori_loop(0, n_chunks, chunk_body, 0)

    return sc_gather


def kernel(feature_values, table):
    batch, seq = feature_values.shape
    flat = feature_values.reshape(-1)
    out = _make_sc_gather(flat.shape[0])(flat, table)
    return out.reshape(batch, seq, EMBED_DIM)


# speedup vs baseline: 1.7700x; 1.0400x over previous
import functools

import jax
import jax.numpy as jnp
from jax import lax
from jax.experimental import pallas as pl
from jax.experimental.pallas import tpu as pltpu
from jax.experimental.pallas import tpu_sc as plsc

NUM_BUCKETS = 1000000
EMBED_DIM = 64
LANES = 16
CHUNK = 320
NBUF = 4


@functools.lru_cache(maxsize=None)
def _make_sc_gather(B: int):
    info = plsc.get_sparse_core_info()
    nc, ns = info.num_cores, info.num_subcores
    nw = nc * ns
    assert B % (8 * nw) == 0
    b_per_w = B // nw
    assert b_per_w % (CHUNK * NBUF) == 0
    n_chunks = b_per_w // CHUNK

    mesh = plsc.VectorSubcoreMesh(core_axis_name="c", subcore_axis_name="s")

    @functools.partial(
        pl.kernel,
        mesh=mesh,
        out_type=jax.ShapeDtypeStruct((B, EMBED_DIM), jnp.float32),
        compiler_params=pltpu.CompilerParams(use_tc_tiling_on_sc=False),
        scratch_types=[
            pltpu.VMEM((b_per_w,), jnp.int32),
            [pltpu.VMEM((CHUNK, EMBED_DIM), jnp.float32) for _ in range(NBUF)],
            [pltpu.SemaphoreType.DMA for _ in range(NBUF)],
        ],
    )
    def sc_gather(fv_hbm, table_hbm, out_hbm, idx_v, rows, gsem):
        wid = lax.axis_index("s") * nc + lax.axis_index("c")
        base = wid * b_per_w

        pltpu.sync_copy(fv_hbm.at[pl.ds(base, b_per_w)], idx_v)

        def mod_body(i, carry):
            v = idx_v[pl.ds(i * LANES, LANES)]
            idx_v[pl.ds(i * LANES, LANES)] = lax.rem(v, NUM_BUCKETS)
            return carry

        lax.fori_loop(0, b_per_w // LANES, mod_body, 0, unroll=8)

        def gather(c, b):
            pltpu.async_copy(
                table_hbm.at[idx_v.at[pl.ds(c * CHUNK, CHUNK)]], rows[b], gsem[b]
            )

        def wait_gather(b):
            pltpu.make_async_copy(
                table_hbm.at[idx_v.at[pl.ds(0, CHUNK)]], rows[b], gsem[b]
            ).wait()

        def writeback(c, b):
            pltpu.sync_copy(rows[b], out_hbm.at[pl.ds(base + c * CHUNK, CHUNK)])

        for b in range(NBUF):
            gather(b, b)

        def ring_body(g, carry):
            for b in range(NBUF):
                c = g * NBUF + b
                wait_gather(b)
                writeback(c, b)
                gather(c + NBUF, b)
            return carry

        lax.fori_loop(0, n_chunks // NBUF - 1, ring_body, 0)

        for b in range(NBUF):
            c = n_chunks - NBUF + b
            wait_gather(b)
            writeback(c, b)

    return sc_gather


def kernel(feature_values, table):
    batch, seq = feature_values.shape
    flat = feature_values.reshape(-1)
    out = _make_sc_gather(flat.shape[0])(flat, table)
    return out.reshape(batch, seq, EMBED_DIM)
